# Initial kernel scaffold; baseline (speedup 1.0000x reference)
import jax
import jax.numpy as jnp
from jax.experimental import pallas as pl


def kernel(x, edge_index, edge_attr, batch, pe, params):
    raise NotImplementedError("write your pallas kernel here")


# speedup vs baseline: 8.2439x; 8.2439x over previous
import functools

import jax
import jax.numpy as jnp
from jax import lax
from jax.experimental import pallas as pl
from jax.experimental.pallas import tpu as pltpu
from jax.experimental.pallas import tpu_sc as plsc

N = 10000
E = 320000
D = 128
L = 4
H = 4
NPG = 50
G = N // NPG
PE_DIM = 8
DH = D // H
FF = 2 * D

BN = 400
NBLK = N // BN

NCORE = 2
NSUB = 16
NW = NCORE * NSUB
EPW = E // NW
CH = 80
NCH = EPW // CH

_SUB_ROWS = N // NSUB
_ROWS_A = 624
_ROWS_B = N - 15 * _ROWS_A


def _sc_agg_body(h_hbm, t_hbm, src_hbm, dst_hbm, code_hbm, z_hbm, out_hbm,
                 idx_s, idx_c, idx_d, hrows, trows, agg_sh, sem1, sem2):
    c_id = lax.axis_index("c")
    s_id = lax.axis_index("s")
    w = c_id * NSUB + s_id

    @pl.when(s_id < NSUB - 1)
    def _():
        pltpu.sync_copy(z_hbm.at[pl.ds(s_id * _ROWS_A, _ROWS_A)],
                        agg_sh.at[pl.ds(s_id * _ROWS_A, _ROWS_A)])

    @pl.when(s_id == NSUB - 1)
    def _():
        pltpu.sync_copy(z_hbm.at[pl.ds(15 * _ROWS_A, _ROWS_B)],
                        agg_sh.at[pl.ds(15 * _ROWS_A, _ROWS_B)])

    plsc.subcore_barrier()

    def chunk(ci, carry):
        base = w * EPW + ci * CH
        pltpu.sync_copy(src_hbm.at[pl.ds(base, CH)], idx_s)
        pltpu.sync_copy(code_hbm.at[pl.ds(base, CH)], idx_c)
        pltpu.sync_copy(dst_hbm.at[pl.ds(base, CH)], idx_d)
        cp1 = pltpu.async_copy(h_hbm.at[idx_s], hrows, sem1)
        cp2 = pltpu.async_copy(t_hbm.at[idx_c], trows, sem2)
        cp1.wait()
        cp2.wait()

        @plsc.parallel_loop(0, CH, unroll=4)
        def _row(i):
            for j in range(D // 16):
                sl = pl.ds(j * 16, 16)
                v = hrows[i, sl] + trows[i, sl]
                hrows[i, sl] = jnp.maximum(v, 0.0)

        pltpu.sync_copy(hrows, agg_sh.at[idx_d], add=True)
        return carry

    lax.fori_loop(0, NCH, chunk, 0)
    plsc.subcore_barrier()

    @pl.when(s_id < NSUB - 1)
    def _():
        pltpu.sync_copy(agg_sh.at[pl.ds(s_id * _ROWS_A, _ROWS_A)],
                        out_hbm.at[c_id, pl.ds(s_id * _ROWS_A, _ROWS_A)])

    @pl.when(s_id == NSUB - 1)
    def _():
        pltpu.sync_copy(agg_sh.at[pl.ds(15 * _ROWS_A, _ROWS_B)],
                        out_hbm.at[c_id, pl.ds(15 * _ROWS_A, _ROWS_B)])


@functools.lru_cache(maxsize=1)
def _get_sc_agg():
  return pl.kernel(
    _sc_agg_body,
    out_type=jax.ShapeDtypeStruct((NCORE, N, D), jnp.float32),
    mesh=plsc.VectorSubcoreMesh(core_axis_name="c", subcore_axis_name="s",
                                num_cores=NCORE, num_subcores=NSUB),
    scratch_types=[
        pltpu.VMEM((CH,), jnp.int32),
        pltpu.VMEM((CH,), jnp.int32),
        pltpu.VMEM((CH,), jnp.int32),
        pltpu.VMEM((CH, D), jnp.float32),
        pltpu.VMEM((CH, D), jnp.float32),
        pltpu.VMEM_SHARED((N, D), jnp.float32),
        pltpu.SemaphoreType.DMA,
        pltpu.SemaphoreType.DMA,
    ],
  )


def _bond_table_body(b_ref, t_ref):
    i = lax.broadcasted_iota(jnp.int32, (512, 8), 0)
    j = lax.broadcasted_iota(jnp.int32, (512, 8), 1)
    oh0 = ((i // 64) == j).astype(jnp.float32)
    oh1 = (((i // 8) % 8) == j).astype(jnp.float32)
    oh2 = ((i % 8) == j).astype(jnp.float32)
    t_ref[...] = oh0 @ b_ref[0] + oh1 @ b_ref[1] + oh2 @ b_ref[2]


def _bond_table(bond_emb):
    return pl.pallas_call(
        _bond_table_body,
        out_shape=jax.ShapeDtypeStruct((512, D), jnp.float32),
    )(bond_emb)


def _encoder_body(x_ref, pe_ref, atom_ref, pew_ref, peb_ref, h_ref):
    xb = x_ref[...]
    acc = pe_ref[...] @ pew_ref[...] + peb_ref[...]
    for f in range(9):
        col = xb[:, f:f + 1]
        oh = (col == lax.broadcasted_iota(jnp.int32, (BN, 64), 1)
              ).astype(jnp.float32)
        acc = acc + oh @ atom_ref[f]
    h_ref[...] = acc


def _encode(x, pe, atom_emb, pe_w, pe_b):
    return pl.pallas_call(
        _encoder_body,
        grid=(NBLK,),
        in_specs=[
            pl.BlockSpec((BN, 9), lambda i: (i, 0)),
            pl.BlockSpec((BN, PE_DIM), lambda i: (i, 0)),
            pl.BlockSpec((9, 64, D), lambda i: (0, 0, 0)),
            pl.BlockSpec((PE_DIM, D), lambda i: (0, 0)),
            pl.BlockSpec((1, D), lambda i: (0, 0)),
        ],
        out_specs=pl.BlockSpec((BN, D), lambda i: (i, 0)),
        out_shape=jax.ShapeDtypeStruct((N, D), jnp.float32),
    )(x, pe, atom_emb, pe_w, pe_b.reshape(1, D))


def _ln(h, g, b):
    mu = jnp.mean(h, axis=-1, keepdims=True)
    var = jnp.mean((h - mu) * (h - mu), axis=-1, keepdims=True)
    return (h - mu) / jnp.sqrt(var + 1e-5) * g + b


def _layer_body(h_ref, agg_ref, eps_ref, gw1_ref, gb1_ref, gw2_ref, gb2_ref,
                wq_ref, wk_ref, wv_ref, wo_ref, ln1g_ref, ln1b_ref,
                ln2g_ref, ln2b_ref, fw1_ref, fb1_ref, fw2_ref, fb2_ref,
                ln3g_ref, ln3b_ref, out_ref):
    h = h_ref[...]
    agg = agg_ref[0] + agg_ref[1]

    loc = (1.0 + eps_ref[0, 0]) * h + agg
    z = jnp.maximum(loc @ gw1_ref[...] + gb1_ref[...], 0.0)
    loc = z @ gw2_ref[...] + gb2_ref[...]
    h_local = _ln(h + loc, ln1g_ref[...], ln1b_ref[...])

    q = h @ wq_ref[...]
    k = h @ wk_ref[...]
    v = h @ wv_ref[...]
    ri = lax.broadcasted_iota(jnp.int32, (BN, BN), 0) // NPG
    ci = lax.broadcasted_iota(jnp.int32, (BN, BN), 1) // NPG
    mask = ri == ci
    scale = 1.0 / (DH ** 0.5)
    outs = []
    for hi in range(H):
        sl = slice(hi * DH, (hi + 1) * DH)
        qh = q[:, sl] * scale
        kh = k[:, sl]
        vh = v[:, sl]
        s = lax.dot_general(qh, kh, (((1,), (1,)), ((), ())))
        s = jnp.where(mask, s, -1e30)
        m = jnp.max(s, axis=-1, keepdims=True)
        e = jnp.exp(s - m)
        p = e / jnp.sum(e, axis=-1, keepdims=True)
        outs.append(p @ vh)
    o = jnp.concatenate(outs, axis=1) @ wo_ref[...]
    h_attn = _ln(h + o, ln2g_ref[...], ln2b_ref[...])

    out = h_local + h_attn
    ffn = jnp.maximum(out @ fw1_ref[...] + fb1_ref[...], 0.0)
    ffn = ffn @ fw2_ref[...] + fb2_ref[...]
    out_ref[...] = _ln(out + ffn, ln3g_ref[...], ln3b_ref[...])


def _layer(h, agg2, eps, gw1, gb1, gw2, gb2, wq, wk, wv, wo,
           ln1g, ln1b, ln2g, ln2b, fw1, fb1, fw2, fb2, ln3g, ln3b):
    full = lambda shape: pl.BlockSpec(shape, lambda i: tuple(0 for _ in shape))
    return pl.pallas_call(
        _layer_body,
        grid=(NBLK,),
        in_specs=[
            pl.BlockSpec((BN, D), lambda i: (i, 0)),
            pl.BlockSpec((NCORE, BN, D), lambda i: (0, i, 0)),
            full((1, 1)),
            full((D, D)), full((1, D)), full((D, D)), full((1, D)),
            full((D, D)), full((D, D)), full((D, D)), full((D, D)),
            full((1, D)), full((1, D)), full((1, D)), full((1, D)),
            full((D, FF)), full((1, FF)), full((FF, D)), full((1, D)),
            full((1, D)), full((1, D)),
        ],
        out_specs=pl.BlockSpec((BN, D), lambda i: (i, 0)),
        out_shape=jax.ShapeDtypeStruct((N, D), jnp.float32),
    )(h, agg2, eps.reshape(1, 1),
      gw1, gb1.reshape(1, D), gw2, gb2.reshape(1, D),
      wq, wk, wv, wo,
      ln1g.reshape(1, D), ln1b.reshape(1, D),
      ln2g.reshape(1, D), ln2b.reshape(1, D),
      fw1, fb1.reshape(1, FF), fw2, fb2.reshape(1, D),
      ln3g.reshape(1, D), ln3b.reshape(1, D))


def _pool_body(h_ref, out_ref):
    gpb = BN // NPG
    oh = (lax.broadcasted_iota(jnp.int32, (gpb, BN), 1) // NPG
          == lax.broadcasted_iota(jnp.int32, (gpb, BN), 0)).astype(jnp.float32)
    out_ref[...] = (oh @ h_ref[...]) * (1.0 / NPG)


def _pool(h):
    gpb = BN // NPG
    return pl.pallas_call(
        _pool_body,
        grid=(NBLK,),
        in_specs=[pl.BlockSpec((BN, D), lambda i: (i, 0))],
        out_specs=pl.BlockSpec((gpb, D), lambda i: (i, 0)),
        out_shape=jax.ShapeDtypeStruct((G, D), jnp.float32),
    )(h)


def _head_body(p_ref, w1_ref, b1_ref, w2_ref, b2_ref, w3_ref, b3_ref, o_ref):
    z = jnp.maximum(p_ref[...] @ w1_ref[...] + b1_ref[...], 0.0)
    z = jnp.maximum(z @ w2_ref[...] + b2_ref[...], 0.0)
    o_ref[...] = z @ w3_ref[...] + b3_ref[...]


def _head(pooled, w1, b1, w2, b2, w3, b3):
    return pl.pallas_call(
        _head_body,
        out_shape=jax.ShapeDtypeStruct((G, 1), jnp.float32),
    )(pooled, w1, b1.reshape(1, D // 2), w2, b2.reshape(1, D // 4),
      w3, b3.reshape(1, 1))


@jax.jit
def _run(x, edge_index, edge_attr, batch, pe, params):
    p = params
    src = edge_index[0].astype(jnp.int32)
    dst = edge_index[1].astype(jnp.int32)
    ea = edge_attr.astype(jnp.int32)
    code = ea[:, 0] * 64 + ea[:, 1] * 8 + ea[:, 2]

    t = _bond_table(p['bond_emb'])
    h = _encode(x.astype(jnp.int32), pe, p['atom_emb'], p['pe_w'], p['pe_b'])
    zeros = jnp.zeros((N, D), jnp.float32)

    for l in range(L):
        agg2 = _get_sc_agg()(h, t, src, dst, code, zeros)
        h = _layer(h, agg2, p['eps'][l],
                   p['gine_w1'][l], p['gine_b1'][l],
                   p['gine_w2'][l], p['gine_b2'][l],
                   p['wq'][l], p['wk'][l], p['wv'][l], p['wo'][l],
                   p['ln1_g'][l], p['ln1_b'][l],
                   p['ln2_g'][l], p['ln2_b'][l],
                   p['ffn_w1'][l], p['ffn_b1'][l],
                   p['ffn_w2'][l], p['ffn_b2'][l],
                   p['ln3_g'][l], p['ln3_b'][l])

    pooled = _pool(h)
    return _head(pooled, p['head_w1'], p['head_b1'],
                 p['head_w2'], p['head_b2'],
                 p['head_w3'], p['head_b3'])


def kernel(x, edge_index, edge_attr, batch, pe, params):
    return _run(x, edge_index, edge_attr, batch, pe, params)


# speedup vs baseline: 12.3154x; 1.4939x over previous
import functools

import jax
import jax.numpy as jnp
from jax import lax
from jax.experimental import pallas as pl
from jax.experimental.pallas import tpu as pltpu
from jax.experimental.pallas import tpu_sc as plsc

N = 10000
E = 320000
D = 128
L = 4
H = 4
NPG = 50
G = N // NPG
PE_DIM = 8
DH = D // H
FF = 2 * D

BN = 400
NBLK = N // BN

NCORE = 2
NSUB = 16
NW = NCORE * NSUB
EPW = E // NW
CH = 80
NCH = EPW // CH

_SUB_ROWS = N // NSUB
_ROWS_A = 624
_ROWS_B = N - 15 * _ROWS_A


def _sc_agg_body(h_hbm, t_hbm, packed_hbm, z_hbm, out_hbm,
                 ib0, ib1, rh0, rh1, rt0, rt1, agg_sh,
                 sem_i0, sem_i1, sem_r0, sem_r1):
    c_id = lax.axis_index("c")
    s_id = lax.axis_index("s")
    w = c_id * NSUB + s_id
    ib = (ib0, ib1)
    rh = (rh0, rh1)
    rt = (rt0, rt1)
    sem_i = (sem_i0, sem_i1)
    sem_r = (sem_r0, sem_r1)

    @pl.when(s_id < NSUB - 1)
    def _():
        pltpu.sync_copy(z_hbm.at[pl.ds(s_id * _ROWS_A, _ROWS_A)],
                        agg_sh.at[pl.ds(s_id * _ROWS_A, _ROWS_A)])

    @pl.when(s_id == NSUB - 1)
    def _():
        pltpu.sync_copy(z_hbm.at[pl.ds(15 * _ROWS_A, _ROWS_B)],
                        agg_sh.at[pl.ds(15 * _ROWS_A, _ROWS_B)])

    plsc.subcore_barrier()

    def issue_idx(ci, b):
        pltpu.async_copy(packed_hbm.at[w, ci], ib[b], sem_i[b])

    def wait_idx(b):
        pltpu.make_async_copy(packed_hbm.at[w, 0], ib[b], sem_i[b]).wait()

    def issue_gathers(b):
        pltpu.async_copy(h_hbm.at[ib[b].at[0]], rh[b], sem_r[b])
        pltpu.async_copy(t_hbm.at[ib[b].at[1]], rt[b], sem_r[b])

    def wait_gathers(b):
        pltpu.make_async_copy(h_hbm.at[ib[b].at[0]], rh[b], sem_r[b]).wait()
        pltpu.make_async_copy(h_hbm.at[ib[b].at[0]], rt[b], sem_r[b]).wait()

    def compute_scatter(b):
        @plsc.parallel_loop(0, CH, unroll=4)
        def _row(i):
            for j in range(D // 16):
                sl = pl.ds(j * 16, 16)
                v = rh[b][i, sl] + rt[b][i, sl]
                rh[b][i, sl] = jnp.maximum(v, 0.0)

        pltpu.sync_copy(rh[b], agg_sh.at[ib[b].at[2]], add=True)

    issue_idx(0, 0)
    issue_idx(1, 1)
    wait_idx(0)
    issue_gathers(0)

    def pair(k, carry):
        c0 = 2 * k
        c1 = c0 + 1
        wait_gathers(0)
        wait_idx(1)
        issue_gathers(1)
        compute_scatter(0)
        issue_idx(c0 + 2, 0)
        wait_gathers(1)
        compute_scatter(1)

        @pl.when(c1 + 2 < NCH)
        def _():
            issue_idx(c1 + 2, 1)

        wait_idx(0)
        issue_gathers(0)
        return carry

    lax.fori_loop(0, (NCH - 1) // 2, pair, 0)
    wait_gathers(0)
    compute_scatter(0)
    plsc.subcore_barrier()

    @pl.when(s_id < NSUB - 1)
    def _():
        pltpu.sync_copy(agg_sh.at[pl.ds(s_id * _ROWS_A, _ROWS_A)],
                        out_hbm.at[c_id, pl.ds(s_id * _ROWS_A, _ROWS_A)])

    @pl.when(s_id == NSUB - 1)
    def _():
        pltpu.sync_copy(agg_sh.at[pl.ds(15 * _ROWS_A, _ROWS_B)],
                        out_hbm.at[c_id, pl.ds(15 * _ROWS_A, _ROWS_B)])


@functools.lru_cache(maxsize=1)
def _get_sc_agg():
  return pl.kernel(
    _sc_agg_body,
    out_type=jax.ShapeDtypeStruct((NCORE, N, D), jnp.float32),
    mesh=plsc.VectorSubcoreMesh(core_axis_name="c", subcore_axis_name="s",
                                num_cores=NCORE, num_subcores=NSUB),
    scratch_types=[
        pltpu.VMEM((3, CH), jnp.int32),
        pltpu.VMEM((3, CH), jnp.int32),
        pltpu.VMEM((CH, D), jnp.float32),
        pltpu.VMEM((CH, D), jnp.float32),
        pltpu.VMEM((CH, D), jnp.float32),
        pltpu.VMEM((CH, D), jnp.float32),
        pltpu.VMEM_SHARED((N, D), jnp.float32),
        pltpu.SemaphoreType.DMA,
        pltpu.SemaphoreType.DMA,
        pltpu.SemaphoreType.DMA,
        pltpu.SemaphoreType.DMA,
    ],
  )


def _bond_table_body(b_ref, t_ref):
    i = lax.broadcasted_iota(jnp.int32, (512, 8), 0)
    j = lax.broadcasted_iota(jnp.int32, (512, 8), 1)
    oh0 = ((i // 64) == j).astype(jnp.float32)
    oh1 = (((i // 8) % 8) == j).astype(jnp.float32)
    oh2 = ((i % 8) == j).astype(jnp.float32)
    t_ref[...] = oh0 @ b_ref[0] + oh1 @ b_ref[1] + oh2 @ b_ref[2]


def _bond_table(bond_emb):
    return pl.pallas_call(
        _bond_table_body,
        out_shape=jax.ShapeDtypeStruct((512, D), jnp.float32),
    )(bond_emb)


def _encoder_body(x_ref, pe_ref, atom_ref, pew_ref, peb_ref, h_ref):
    xb = x_ref[...]
    acc = pe_ref[...] @ pew_ref[...] + peb_ref[...]
    for f in range(9):
        col = xb[:, f:f + 1]
        oh = (col == lax.broadcasted_iota(jnp.int32, (BN, 64), 1)
              ).astype(jnp.float32)
        acc = acc + oh @ atom_ref[f]
    h_ref[...] = acc


def _encode(x, pe, atom_emb, pe_w, pe_b):
    return pl.pallas_call(
        _encoder_body,
        grid=(NBLK,),
        in_specs=[
            pl.BlockSpec((BN, 9), lambda i: (i, 0)),
            pl.BlockSpec((BN, PE_DIM), lambda i: (i, 0)),
            pl.BlockSpec((9, 64, D), lambda i: (0, 0, 0)),
            pl.BlockSpec((PE_DIM, D), lambda i: (0, 0)),
            pl.BlockSpec((1, D), lambda i: (0, 0)),
        ],
        out_specs=pl.BlockSpec((BN, D), lambda i: (i, 0)),
        out_shape=jax.ShapeDtypeStruct((N, D), jnp.float32),
    )(x, pe, atom_emb, pe_w, pe_b.reshape(1, D))


def _ln(h, g, b):
    mu = jnp.mean(h, axis=-1, keepdims=True)
    var = jnp.mean((h - mu) * (h - mu), axis=-1, keepdims=True)
    return (h - mu) / jnp.sqrt(var + 1e-5) * g + b


def _layer_body(h_ref, agg_ref, eps_ref, gw1_ref, gb1_ref, gw2_ref, gb2_ref,
                wq_ref, wk_ref, wv_ref, wo_ref, ln1g_ref, ln1b_ref,
                ln2g_ref, ln2b_ref, fw1_ref, fb1_ref, fw2_ref, fb2_ref,
                ln3g_ref, ln3b_ref, out_ref):
    h = h_ref[...]
    agg = agg_ref[0] + agg_ref[1]

    loc = (1.0 + eps_ref[0, 0]) * h + agg
    z = jnp.maximum(loc @ gw1_ref[...] + gb1_ref[...], 0.0)
    loc = z @ gw2_ref[...] + gb2_ref[...]
    h_local = _ln(h + loc, ln1g_ref[...], ln1b_ref[...])

    q = h @ wq_ref[...]
    k = h @ wk_ref[...]
    v = h @ wv_ref[...]
    ri = lax.broadcasted_iota(jnp.int32, (BN, BN), 0) // NPG
    ci = lax.broadcasted_iota(jnp.int32, (BN, BN), 1) // NPG
    mask = ri == ci
    scale = 1.0 / (DH ** 0.5)
    outs = []
    for hi in range(H):
        sl = slice(hi * DH, (hi + 1) * DH)
        qh = q[:, sl] * scale
        kh = k[:, sl]
        vh = v[:, sl]
        s = lax.dot_general(qh, kh, (((1,), (1,)), ((), ())))
        s = jnp.where(mask, s, -1e30)
        m = jnp.max(s, axis=-1, keepdims=True)
        e = jnp.exp(s - m)
        p = e / jnp.sum(e, axis=-1, keepdims=True)
        outs.append(p @ vh)
    o = jnp.concatenate(outs, axis=1) @ wo_ref[...]
    h_attn = _ln(h + o, ln2g_ref[...], ln2b_ref[...])

    out = h_local + h_attn
    ffn = jnp.maximum(out @ fw1_ref[...] + fb1_ref[...], 0.0)
    ffn = ffn @ fw2_ref[...] + fb2_ref[...]
    out_ref[...] = _ln(out + ffn, ln3g_ref[...], ln3b_ref[...])


def _layer(h, agg2, eps, gw1, gb1, gw2, gb2, wq, wk, wv, wo,
           ln1g, ln1b, ln2g, ln2b, fw1, fb1, fw2, fb2, ln3g, ln3b):
    full = lambda shape: pl.BlockSpec(shape, lambda i: tuple(0 for _ in shape))
    return pl.pallas_call(
        _layer_body,
        grid=(NBLK,),
        in_specs=[
            pl.BlockSpec((BN, D), lambda i: (i, 0)),
            pl.BlockSpec((NCORE, BN, D), lambda i: (0, i, 0)),
            full((1, 1)),
            full((D, D)), full((1, D)), full((D, D)), full((1, D)),
            full((D, D)), full((D, D)), full((D, D)), full((D, D)),
            full((1, D)), full((1, D)), full((1, D)), full((1, D)),
            full((D, FF)), full((1, FF)), full((FF, D)), full((1, D)),
            full((1, D)), full((1, D)),
        ],
        out_specs=pl.BlockSpec((BN, D), lambda i: (i, 0)),
        out_shape=jax.ShapeDtypeStruct((N, D), jnp.float32),
    )(h, agg2, eps.reshape(1, 1),
      gw1, gb1.reshape(1, D), gw2, gb2.reshape(1, D),
      wq, wk, wv, wo,
      ln1g.reshape(1, D), ln1b.reshape(1, D),
      ln2g.reshape(1, D), ln2b.reshape(1, D),
      fw1, fb1.reshape(1, FF), fw2, fb2.reshape(1, D),
      ln3g.reshape(1, D), ln3b.reshape(1, D))


def _pool_body(h_ref, out_ref):
    gpb = BN // NPG
    oh = (lax.broadcasted_iota(jnp.int32, (gpb, BN), 1) // NPG
          == lax.broadcasted_iota(jnp.int32, (gpb, BN), 0)).astype(jnp.float32)
    out_ref[...] = (oh @ h_ref[...]) * (1.0 / NPG)


def _pool(h):
    gpb = BN // NPG
    return pl.pallas_call(
        _pool_body,
        grid=(NBLK,),
        in_specs=[pl.BlockSpec((BN, D), lambda i: (i, 0))],
        out_specs=pl.BlockSpec((gpb, D), lambda i: (i, 0)),
        out_shape=jax.ShapeDtypeStruct((G, D), jnp.float32),
    )(h)


def _head_body(p_ref, w1_ref, b1_ref, w2_ref, b2_ref, w3_ref, b3_ref, o_ref):
    z = jnp.maximum(p_ref[...] @ w1_ref[...] + b1_ref[...], 0.0)
    z = jnp.maximum(z @ w2_ref[...] + b2_ref[...], 0.0)
    o_ref[...] = z @ w3_ref[...] + b3_ref[...]


def _head(pooled, w1, b1, w2, b2, w3, b3):
    return pl.pallas_call(
        _head_body,
        out_shape=jax.ShapeDtypeStruct((G, 1), jnp.float32),
    )(pooled, w1, b1.reshape(1, D // 2), w2, b2.reshape(1, D // 4),
      w3, b3.reshape(1, 1))


@jax.jit
def _run(x, edge_index, edge_attr, batch, pe, params):
    p = params
    src = edge_index[0].astype(jnp.int32)
    dst = edge_index[1].astype(jnp.int32)
    ea = edge_attr.astype(jnp.int32)
    code = ea[:, 0] * 64 + ea[:, 1] * 8 + ea[:, 2]
    packed = (jnp.stack([src, code, dst], axis=0)
              .reshape(3, NW, NCH, CH).transpose(1, 2, 0, 3))

    t = _bond_table(p['bond_emb'])
    h = _encode(x.astype(jnp.int32), pe, p['atom_emb'], p['pe_w'], p['pe_b'])
    zeros = jnp.zeros((N, D), jnp.float32)

    for l in range(L):
        agg2 = _get_sc_agg()(h, t, packed, zeros)
        h = _layer(h, agg2, p['eps'][l],
                   p['gine_w1'][l], p['gine_b1'][l],
                   p['gine_w2'][l], p['gine_b2'][l],
                   p['wq'][l], p['wk'][l], p['wv'][l], p['wo'][l],
                   p['ln1_g'][l], p['ln1_b'][l],
                   p['ln2_g'][l], p['ln2_b'][l],
                   p['ffn_w1'][l], p['ffn_b1'][l],
                   p['ffn_w2'][l], p['ffn_b2'][l],
                   p['ln3_g'][l], p['ln3_b'][l])

    pooled = _pool(h)
    return _head(pooled, p['head_w1'], p['head_b1'],
                 p['head_w2'], p['head_b2'],
                 p['head_w3'], p['head_b3'])


def kernel(x, edge_index, edge_attr, batch, pe, params):
    return _run(x, edge_index, edge_attr, batch, pe, params)
